# baseline (speedup 1.0000x reference)
import jax
import jax.numpy as jnp
from jax import lax
from jax.experimental import pallas as pl
from jax.experimental.pallas import tpu as pltpu
from jax.experimental.pallas import tpu_sc as plsc

B = 8
N = 8192
S = 256
K = 32

_NC = 2
_NS = 16
_NW = _NC * _NS
_ROWS = B * S
_RPW = _ROWS // _NW
_CAND = 256
_VPR = N // 16


def _fps_body(x_ref, y_ref, z_ref, nx_ref, ny_ref, nz_ref):
    X = x_ref[...]
    Y = y_ref[...]
    Z = z_ref[...]
    col = lax.broadcasted_iota(jnp.int32, (B, N), 1)
    col_s = lax.broadcasted_iota(jnp.int32, (B, S), 1)

    def step(i, carry):
        dist, f, ax, ay, az = carry
        onehot = col == f
        cx = jnp.sum(jnp.where(onehot, X, 0.0), axis=1, keepdims=True)
        cy = jnp.sum(jnp.where(onehot, Y, 0.0), axis=1, keepdims=True)
        cz = jnp.sum(jnp.where(onehot, Z, 0.0), axis=1, keepdims=True)
        sel = col_s == i
        ax = jnp.where(sel, cx, ax)
        ay = jnp.where(sel, cy, ay)
        az = jnp.where(sel, cz, az)
        d = (X - cx) ** 2 + (Y - cy) ** 2 + (Z - cz) ** 2
        dist = jnp.minimum(dist, d)
        m = jnp.max(dist, axis=1, keepdims=True)
        f = jnp.min(jnp.where(dist == m, col, N), axis=1, keepdims=True)
        return dist, f, ax, ay, az

    zero_s = jnp.zeros((B, S), jnp.float32)
    init = (jnp.full((B, N), 1e10, jnp.float32),
            jnp.zeros((B, 1), jnp.int32), zero_s, zero_s, zero_s)
    _, _, ax, ay, az = lax.fori_loop(0, S, step, init)
    nx_ref[...] = ax
    ny_ref[...] = ay
    nz_ref[...] = az


def _fps(x, y, z):
    out = jax.ShapeDtypeStruct((B, S), jnp.float32)
    return pl.pallas_call(
        _fps_body,
        out_shape=[out, out, out],
    )(x, y, z)


def _feat_body(xyz8_ref, pts_ref, q8_ref, w1a_ref, w1b_ref, b1_ref,
               w2_ref, b2_ref, dd_ref, tau_ref, h2_ref):
    xyz8 = xyz8_ref[0]
    q8 = q8_ref[0]
    inner = jnp.dot(q8, xyz8, preferred_element_type=jnp.float32)
    x2 = jnp.sum(xyz8 * xyz8, axis=0, keepdims=True)
    q2 = jnp.sum(q8 * q8, axis=1, keepdims=True)
    dd = (q2 + x2) - 2.0 * inner
    dd_ref[0] = dd

    w = dd
    for width in (4096, 2048, 1024, 512, 256, 128):
        w = jnp.minimum(w[:, :width], w[:, width:2 * width])
    colg = lax.broadcasted_iota(jnp.int32, (S, 128), 1)

    def tsel(_, wc):
        m = jnp.min(wc, axis=1, keepdims=True)
        p = jnp.min(jnp.where(wc == m, colg, 128), axis=1, keepdims=True)
        return jnp.where(colg == p, jnp.inf, wc)

    w31 = lax.fori_loop(0, K - 1, tsel, w)
    tau = jnp.min(w31, axis=1, keepdims=True)
    tau_ref[0] = jnp.broadcast_to(tau, (S, 16))

    pts = pts_ref[0]
    h1 = (jnp.dot(w1a_ref[...], xyz8, preferred_element_type=jnp.float32)
          + jnp.dot(w1b_ref[...], pts, preferred_element_type=jnp.float32))
    h1 = jnp.maximum(h1 + b1_ref[...], 0.0)
    h2 = jnp.dot(w2_ref[...], h1, preferred_element_type=jnp.float32)
    h2 = jnp.maximum(h2 + b2_ref[...], 0.0)
    h2_ref[0, :, 0:64] = h2.T


def _feat(xyz8, pts, q8, w1a, w1b, b1c, w2, b2c):
    return pl.pallas_call(
        _feat_body,
        grid=(B,),
        in_specs=[
            pl.BlockSpec((1, 8, N), lambda b: (b, 0, 0)),
            pl.BlockSpec((1, 32, N), lambda b: (b, 0, 0)),
            pl.BlockSpec((1, S, 8), lambda b: (b, 0, 0)),
            pl.BlockSpec((32, 8), lambda b: (0, 0)),
            pl.BlockSpec((32, 32), lambda b: (0, 0)),
            pl.BlockSpec((32, 1), lambda b: (0, 0)),
            pl.BlockSpec((64, 32), lambda b: (0, 0)),
            pl.BlockSpec((64, 1), lambda b: (0, 0)),
        ],
        out_specs=[
            pl.BlockSpec((1, S, N), lambda b: (b, 0, 0)),
            pl.BlockSpec((1, S, 16), lambda b: (b, 0, 0)),
            pl.BlockSpec((1, N, 128), lambda b: (b, 0, 0)),
        ],
        out_shape=[
            jax.ShapeDtypeStruct((B, S, N), jnp.float32),
            jax.ShapeDtypeStruct((B, S, 16), jnp.float32),
            jax.ShapeDtypeStruct((B, N, 128), jnp.float32),
        ],
    )(xyz8, pts, q8, w1a, w1b, b1c, w2, b2c)


def _rev(x):
    return lax.rev(x, (0,))


def _minmax_kv(ak, av, bk, bv):
    m = ak <= bk
    return (jnp.where(m, ak, bk), jnp.where(m, av, bv),
            jnp.where(m, bk, ak), jnp.where(m, bv, av))


def _merge16(ak, av, bk, bv):
    brk, brv = _rev(bk), _rev(bv)
    lok, lov, hik, hiv = _minmax_kv(ak, av, brk, brv)
    lok, lov = plsc.sort_key_val(lok, lov)
    hik, hiv = plsc.sort_key_val(hik, hiv)
    return lok, lov, hik, hiv


def _merge32_keep32(x1k, x1v, x2k, x2v, y1k, y1v, y2k, y2v):
    ry1k, ry1v = _rev(y1k), _rev(y1v)
    ry2k, ry2v = _rev(y2k), _rev(y2v)
    lo1k, lo1v, _, _ = _minmax_kv(x1k, x1v, ry2k, ry2v)
    lo2k, lo2v, _, _ = _minmax_kv(x2k, x2v, ry1k, ry1v)
    m1k, m1v, m2k, m2v = _minmax_kv(lo1k, lo1v, lo2k, lo2v)
    m1k, m1v = plsc.sort_key_val(m1k, m1v)
    m2k, m2v = plsc.sort_key_val(m2k, m2v)
    return m1k, m1v, m2k, m2v


_RSEG = 4
_SEGW = N // _RSEG
_CAPR = _CAND // _RSEG


def _sc_topk_body(dd_hbm, tau_hbm, h2_hbm, out_hbm,
                  row_a, row_b, kb, vb, taub, idx_a, idx_b,
                  rows_a, rows_b, res_v, sem_a, sem_b, sem_g):
    cid = lax.axis_index("c")
    sid = lax.axis_index("s")
    wid = sid * _NC + cid
    base = wid * _RPW
    pltpu.sync_copy(tau_hbm.at[pl.ds(base * 16, _RPW * 16)], taub)
    pltpu.async_copy(dd_hbm.at[pl.ds(base, 1)], row_a, sem_a)
    pltpu.async_copy(dd_hbm.at[pl.ds(base + 1, 1)], row_b, sem_b)
    lane = lax.iota(jnp.int32, 16)
    inf16 = jnp.full((16,), jnp.inf, jnp.float32)

    def extract(rl, row_v):
        for i in range(_CAND // 16):
            kb[pl.ds(i * 16, 16)] = inf16
        tauv = taub[pl.ds(rl * 16, 16)]
        rb = ((base + rl) // S) * N

        def ext(j, carry):
            offs, ivecs = carry
            n_offs, n_ivecs = [], []
            for r in range(_RSEG):
                s0 = r * _SEGW + j * 32
                v1 = row_v[0, pl.ds(s0, 16)]
                v2 = row_v[0, pl.ds(s0 + 16, 16)]
                m1 = v1 <= tauv
                m2 = v2 <= tauv
                anyv = plsc.all_reduce_population_count(m1 | m2)
                lim = (r + 1) * _CAPR - 16
                iv = ivecs[r]

                def hit(off_r, v1=v1, v2=v2, m1=m1, m2=m2, iv=iv, lim=lim):
                    off_c = jnp.minimum(off_r, lim)
                    plsc.store_compressed(kb.at[pl.ds(off_c, 16)], v1,
                                          mask=m1)
                    plsc.store_compressed(vb.at[pl.ds(off_c, 16)], iv,
                                          mask=m1)
                    o2 = off_r + plsc.all_reduce_population_count(m1)[0]
                    off_c2 = jnp.minimum(o2, lim)
                    plsc.store_compressed(kb.at[pl.ds(off_c2, 16)], v2,
                                          mask=m2)
                    plsc.store_compressed(vb.at[pl.ds(off_c2, 16)], iv + 16,
                                          mask=m2)
                    return o2 + plsc.all_reduce_population_count(m2)[0]

                n_offs.append(
                    lax.cond(anyv[0] > 0, hit, lambda o: o, offs[r]))
                n_ivecs.append(iv + 32)
            return tuple(n_offs), tuple(n_ivecs)

        offs0 = tuple(jnp.int32(r * _CAPR) for r in range(_RSEG))
        ivecs0 = tuple(lane + rb + r * _SEGW for r in range(_RSEG))
        offs, _ = lax.fori_loop(0, _SEGW // 32, ext, (offs0, ivecs0))
        return offs

    def select(par_idxb, offs):
        def sortpair(s0):
            a_k, a_v = plsc.sort_key_val(kb[pl.ds(s0, 16)],
                                         vb[pl.ds(s0, 16)])
            b_k, b_v = plsc.sort_key_val(kb[pl.ds(s0 + 16, 16)],
                                         vb[pl.ds(s0 + 16, 16)])
            return _merge16(a_k, a_v, b_k, b_v)

        bf = sortpair(0)
        for r in range(1, _RSEG):
            bf = _merge32_keep32(*bf, *sortpair(r * _CAPR))
        for r in range(_RSEG):
            cnt_r = offs[r] - r * _CAPR
            bf = lax.cond(
                cnt_r > 32,
                lambda bf=bf, r=r: _merge32_keep32(
                    *bf, *sortpair(r * _CAPR + 32)),
                lambda bf=bf: bf)
        par_idxb[pl.ds(0, 16)] = bf[1]
        par_idxb[pl.ds(16, 16)] = bf[3]

    def reduce_into(r_prev, rows_v):
        for cb in range(4):
            sl = pl.ds(cb * 16, 16)
            acc = rows_v[0, sl]
            for r in range(1, K):
                acc = jnp.maximum(acc, rows_v[r, sl])
            res_v[r_prev, sl] = acc

    bufs = ((row_a, sem_a, idx_a, rows_a), (row_b, sem_b, idx_b, rows_b))

    def pair_loop(rp, _):
        for par in range(2):
            rl = rp * 2 + par
            row_v, sem, idxb, rows_v = bufs[par]
            o_idxb, o_rows = bufs[1 - par][2], bufs[1 - par][3]
            pltpu.make_async_copy(
                dd_hbm.at[pl.ds(base + rl, 1)], row_v, sem).wait()
            offs = extract(rl, row_v)

            @pl.when(rl + 2 < _RPW)
            def _prefetch():
                pltpu.async_copy(
                    dd_hbm.at[pl.ds(base + rl + 2, 1)], row_v, sem)
            if par == 1:
                pltpu.make_async_copy(
                    h2_hbm.at[o_idxb], o_rows, sem_g).wait()
                reduce_into(rl - 1, o_rows)
            else:
                @pl.when(rp > 0)
                def _drain():
                    pltpu.make_async_copy(
                        h2_hbm.at[o_idxb], o_rows, sem_g).wait()
                    reduce_into(rl - 1, o_rows)
            select(idxb, offs)
            pltpu.async_copy(h2_hbm.at[idxb], rows_v, sem_g)
        return 0

    lax.fori_loop(0, _RPW // 2, pair_loop, 0)
    pltpu.make_async_copy(h2_hbm.at[idx_b], rows_b, sem_g).wait()
    reduce_into(_RPW - 1, rows_b)
    pltpu.sync_copy(res_v, out_hbm.at[pl.ds(base, _RPW)])


def _sc_topk_gather_max(ddf, tauf, h2f):
    run = pl.kernel(
        _sc_topk_body,
        out_type=jax.ShapeDtypeStruct((_ROWS, 64), jnp.float32),
        mesh=plsc.VectorSubcoreMesh(
            core_axis_name="c", subcore_axis_name="s",
            num_cores=_NC, num_subcores=_NS),
        scratch_types=[
            pltpu.VMEM((1, N), jnp.float32),
            pltpu.VMEM((1, N), jnp.float32),
            pltpu.VMEM((_CAND,), jnp.float32),
            pltpu.VMEM((_CAND,), jnp.int32),
            pltpu.VMEM((_RPW * 16,), jnp.float32),
            pltpu.VMEM((K,), jnp.int32),
            pltpu.VMEM((K,), jnp.int32),
            pltpu.VMEM((K, 128), jnp.float32),
            pltpu.VMEM((K, 128), jnp.float32),
            pltpu.VMEM((_RPW, 64), jnp.float32),
            pltpu.SemaphoreType.DMA,
            pltpu.SemaphoreType.DMA,
            pltpu.SemaphoreType.DMA,
        ],
        compiler_params=pltpu.CompilerParams(needs_layout_passes=False),
    )
    return run(ddf, tauf, h2f)


def kernel(xyz, points, W1, b1, W2, b2):
    x = xyz[:, :, 0]
    y = xyz[:, :, 1]
    z = xyz[:, :, 2]
    nx, ny, nz = _fps(x, y, z)
    new_xyz = jnp.stack([nx, ny, nz], axis=-1)

    xyz_t = jnp.transpose(xyz, (0, 2, 1))
    xyz8 = jnp.concatenate(
        [xyz_t, jnp.zeros((B, 5, N), jnp.float32)], axis=1)
    q8 = jnp.concatenate(
        [jnp.stack([nx, ny, nz], axis=-1),
         jnp.zeros((B, S, 5), jnp.float32)], axis=-1)
    w1a = jnp.concatenate([W1[:, :3], jnp.zeros((32, 5), jnp.float32)], axis=1)
    w1b = W1[:, 3:]
    dd, tau, h2 = _feat(xyz8, points, q8, w1a, w1b,
                        b1.reshape(32, 1), W2, b2.reshape(64, 1))

    ddf = dd.reshape(_ROWS, N)
    tauf = tau.reshape(_ROWS * 16)
    h2f = h2.reshape(B * N, 128)
    pooled = _sc_topk_gather_max(ddf, tauf, h2f)
    new_points = jnp.transpose(pooled.reshape(B, S, 64), (0, 2, 1))
    return new_xyz, new_points


# speedup vs baseline: 1.0320x; 1.0320x over previous
import jax
import jax.numpy as jnp
from jax import lax
from jax.experimental import pallas as pl
from jax.experimental.pallas import tpu as pltpu
from jax.experimental.pallas import tpu_sc as plsc

B = 8
N = 8192
S = 256
K = 32

_NC = 2
_NS = 16
_NW = _NC * _NS
_ROWS = B * S
_RPW = _ROWS // _NW
_CAND = 256
_VPR = N // 16


def _fps_body(x_ref, y_ref, z_ref, nx_ref, ny_ref, nz_ref):
    X = x_ref[...]
    Y = y_ref[...]
    Z = z_ref[...]
    col = lax.broadcasted_iota(jnp.int32, (B, N), 1)
    col_s = lax.broadcasted_iota(jnp.int32, (B, S), 1)

    def step(i, carry):
        dist, f, ax, ay, az = carry
        onehot = col == f
        cx = jnp.sum(jnp.where(onehot, X, 0.0), axis=1, keepdims=True)
        cy = jnp.sum(jnp.where(onehot, Y, 0.0), axis=1, keepdims=True)
        cz = jnp.sum(jnp.where(onehot, Z, 0.0), axis=1, keepdims=True)
        sel = col_s == i
        ax = jnp.where(sel, cx, ax)
        ay = jnp.where(sel, cy, ay)
        az = jnp.where(sel, cz, az)
        d = (X - cx) ** 2 + (Y - cy) ** 2 + (Z - cz) ** 2
        dist = jnp.minimum(dist, d)
        m = jnp.max(dist, axis=1, keepdims=True)
        f = jnp.min(jnp.where(dist == m, col, N), axis=1, keepdims=True)
        return dist, f, ax, ay, az

    zero_s = jnp.zeros((B, S), jnp.float32)
    init = (jnp.full((B, N), 1e10, jnp.float32),
            jnp.zeros((B, 1), jnp.int32), zero_s, zero_s, zero_s)
    _, _, ax, ay, az = lax.fori_loop(0, S, step, init)
    nx_ref[...] = ax
    ny_ref[...] = ay
    nz_ref[...] = az


def _fps(x, y, z):
    out = jax.ShapeDtypeStruct((B, S), jnp.float32)
    return pl.pallas_call(
        _fps_body,
        out_shape=[out, out, out],
    )(x, y, z)


def _feat_body(xyz8_ref, pts_ref, q8_ref, w1a_ref, w1b_ref, b1_ref,
               w2_ref, b2_ref, dd_ref, tau_ref, h2_ref):
    xyz8 = xyz8_ref[0]
    q8 = q8_ref[0]
    inner = jnp.dot(q8, xyz8, preferred_element_type=jnp.float32)
    x2 = jnp.sum(xyz8 * xyz8, axis=0, keepdims=True)
    q2 = jnp.sum(q8 * q8, axis=1, keepdims=True)
    dd = (q2 + x2) - 2.0 * inner
    dd_ref[0] = dd

    w = dd
    for width in (4096, 2048, 1024, 512, 256, 128):
        w = jnp.minimum(w[:, :width], w[:, width:2 * width])
    colg = lax.broadcasted_iota(jnp.int32, (S, 128), 1)

    def tsel(_, wc):
        m = jnp.min(wc, axis=1, keepdims=True)
        p = jnp.min(jnp.where(wc == m, colg, 128), axis=1, keepdims=True)
        return jnp.where(colg == p, jnp.inf, wc)

    w31 = lax.fori_loop(0, K - 1, tsel, w)
    tau = jnp.min(w31, axis=1, keepdims=True)
    tau_ref[0] = jnp.broadcast_to(tau, (S, 16))

    pts = pts_ref[0]
    h1 = (jnp.dot(w1a_ref[...], xyz8, preferred_element_type=jnp.float32)
          + jnp.dot(w1b_ref[...], pts, preferred_element_type=jnp.float32))
    h1 = jnp.maximum(h1 + b1_ref[...], 0.0)
    h2 = jnp.dot(w2_ref[...], h1, preferred_element_type=jnp.float32)
    h2 = jnp.maximum(h2 + b2_ref[...], 0.0)
    h2_ref[0, :, 0:64] = h2.T


def _feat(xyz8, pts, q8, w1a, w1b, b1c, w2, b2c):
    return pl.pallas_call(
        _feat_body,
        grid=(B,),
        in_specs=[
            pl.BlockSpec((1, 8, N), lambda b: (b, 0, 0)),
            pl.BlockSpec((1, 32, N), lambda b: (b, 0, 0)),
            pl.BlockSpec((1, S, 8), lambda b: (b, 0, 0)),
            pl.BlockSpec((32, 8), lambda b: (0, 0)),
            pl.BlockSpec((32, 32), lambda b: (0, 0)),
            pl.BlockSpec((32, 1), lambda b: (0, 0)),
            pl.BlockSpec((64, 32), lambda b: (0, 0)),
            pl.BlockSpec((64, 1), lambda b: (0, 0)),
        ],
        out_specs=[
            pl.BlockSpec((1, S, N), lambda b: (b, 0, 0)),
            pl.BlockSpec((1, S, 16), lambda b: (b, 0, 0)),
            pl.BlockSpec((1, N, 128), lambda b: (b, 0, 0)),
        ],
        out_shape=[
            jax.ShapeDtypeStruct((B, S, N), jnp.float32),
            jax.ShapeDtypeStruct((B, S, 16), jnp.float32),
            jax.ShapeDtypeStruct((B, N, 128), jnp.float32),
        ],
    )(xyz8, pts, q8, w1a, w1b, b1c, w2, b2c)


def _rev(x):
    return lax.rev(x, (0,))


def _minmax_kv(ak, av, bk, bv):
    m = ak <= bk
    return (jnp.where(m, ak, bk), jnp.where(m, av, bv),
            jnp.where(m, bk, ak), jnp.where(m, bv, av))


def _merge16(ak, av, bk, bv):
    brk, brv = _rev(bk), _rev(bv)
    lok, lov, hik, hiv = _minmax_kv(ak, av, brk, brv)
    lok, lov = plsc.sort_key_val(lok, lov)
    hik, hiv = plsc.sort_key_val(hik, hiv)
    return lok, lov, hik, hiv


def _merge32_keep32(x1k, x1v, x2k, x2v, y1k, y1v, y2k, y2v):
    ry1k, ry1v = _rev(y1k), _rev(y1v)
    ry2k, ry2v = _rev(y2k), _rev(y2v)
    lo1k, lo1v, _, _ = _minmax_kv(x1k, x1v, ry2k, ry2v)
    lo2k, lo2v, _, _ = _minmax_kv(x2k, x2v, ry1k, ry1v)
    m1k, m1v, m2k, m2v = _minmax_kv(lo1k, lo1v, lo2k, lo2v)
    m1k, m1v = plsc.sort_key_val(m1k, m1v)
    m2k, m2v = plsc.sort_key_val(m2k, m2v)
    return m1k, m1v, m2k, m2v


_RSEG = 4
_SEGW = N // _RSEG
_CAPR = _CAND // _RSEG


def _sc_topk_body(dd_hbm, tau_hbm, h2_hbm, out_hbm,
                  row_a, row_b, kb, vb, taub, idx_a, idx_b,
                  rows_a, rows_b, res_v, sem_a, sem_b, sem_g):
    cid = lax.axis_index("c")
    sid = lax.axis_index("s")
    wid = sid * _NC + cid
    base = wid * _RPW
    pltpu.sync_copy(tau_hbm.at[pl.ds(base * 16, _RPW * 16)], taub)
    pltpu.async_copy(dd_hbm.at[pl.ds(base, 1)], row_a, sem_a)
    pltpu.async_copy(dd_hbm.at[pl.ds(base + 1, 1)], row_b, sem_b)
    lane = lax.iota(jnp.int32, 16)
    inf16 = jnp.full((16,), jnp.inf, jnp.float32)

    def extract(rl, row_v):
        for i in range(_CAND // 16):
            kb[pl.ds(i * 16, 16)] = inf16
        tauv = taub[pl.ds(rl * 16, 16)]
        rb = ((base + rl) // S) * N

        def ext(j, carry):
            offs, ivecs = carry
            n_offs, n_ivecs = [], []
            for r in range(_RSEG):
                v = row_v[0, pl.ds(r * _SEGW + j * 16, 16)]
                msk = v <= tauv
                ones = jnp.where(msk, 1, 0).astype(jnp.int32)
                c = plsc.cumsum(ones)
                pos = offs[r] + c - 1
                m2 = msk & (pos < (r + 1) * _CAPR)
                plsc.store_scatter(kb, [pos], v, mask=m2)
                plsc.store_scatter(vb, [pos], ivecs[r], mask=m2)
                cntv = plsc.all_reduce_population_count(msk)
                n_offs.append(offs[r] + cntv)
                n_ivecs.append(ivecs[r] + 16)
            return tuple(n_offs), tuple(n_ivecs)

        offs0 = tuple(jnp.full((16,), r * _CAPR, jnp.int32)
                      for r in range(_RSEG))
        ivecs0 = tuple(lane + rb + r * _SEGW for r in range(_RSEG))
        offs, _ = lax.fori_loop(0, _SEGW // 16, ext, (offs0, ivecs0))
        return tuple(o[0] for o in offs)

    def select(par_idxb, offs):
        def sortpair(s0):
            a_k, a_v = plsc.sort_key_val(kb[pl.ds(s0, 16)],
                                         vb[pl.ds(s0, 16)])
            b_k, b_v = plsc.sort_key_val(kb[pl.ds(s0 + 16, 16)],
                                         vb[pl.ds(s0 + 16, 16)])
            return _merge16(a_k, a_v, b_k, b_v)

        bf = sortpair(0)
        for r in range(1, _RSEG):
            bf = _merge32_keep32(*bf, *sortpair(r * _CAPR))
        for r in range(_RSEG):
            cnt_r = offs[r] - r * _CAPR
            bf = lax.cond(
                cnt_r > 32,
                lambda bf=bf, r=r: _merge32_keep32(
                    *bf, *sortpair(r * _CAPR + 32)),
                lambda bf=bf: bf)
        par_idxb[pl.ds(0, 16)] = bf[1]
        par_idxb[pl.ds(16, 16)] = bf[3]

    def reduce_into(r_prev, rows_v):
        for cb in range(4):
            sl = pl.ds(cb * 16, 16)
            acc = rows_v[0, sl]
            for r in range(1, K):
                acc = jnp.maximum(acc, rows_v[r, sl])
            res_v[r_prev, sl] = acc

    bufs = ((row_a, sem_a, idx_a, rows_a), (row_b, sem_b, idx_b, rows_b))

    def pair_loop(rp, _):
        for par in range(2):
            rl = rp * 2 + par
            row_v, sem, idxb, rows_v = bufs[par]
            o_idxb, o_rows = bufs[1 - par][2], bufs[1 - par][3]
            pltpu.make_async_copy(
                dd_hbm.at[pl.ds(base + rl, 1)], row_v, sem).wait()
            offs = extract(rl, row_v)

            @pl.when(rl + 2 < _RPW)
            def _prefetch():
                pltpu.async_copy(
                    dd_hbm.at[pl.ds(base + rl + 2, 1)], row_v, sem)
            if par == 1:
                pltpu.make_async_copy(
                    h2_hbm.at[o_idxb], o_rows, sem_g).wait()
                reduce_into(rl - 1, o_rows)
            else:
                @pl.when(rp > 0)
                def _drain():
                    pltpu.make_async_copy(
                        h2_hbm.at[o_idxb], o_rows, sem_g).wait()
                    reduce_into(rl - 1, o_rows)
            select(idxb, offs)
            pltpu.async_copy(h2_hbm.at[idxb], rows_v, sem_g)
        return 0

    lax.fori_loop(0, _RPW // 2, pair_loop, 0)
    pltpu.make_async_copy(h2_hbm.at[idx_b], rows_b, sem_g).wait()
    reduce_into(_RPW - 1, rows_b)
    pltpu.sync_copy(res_v, out_hbm.at[pl.ds(base, _RPW)])


def _sc_topk_gather_max(ddf, tauf, h2f):
    run = pl.kernel(
        _sc_topk_body,
        out_type=jax.ShapeDtypeStruct((_ROWS, 64), jnp.float32),
        mesh=plsc.VectorSubcoreMesh(
            core_axis_name="c", subcore_axis_name="s",
            num_cores=_NC, num_subcores=_NS),
        scratch_types=[
            pltpu.VMEM((1, N), jnp.float32),
            pltpu.VMEM((1, N), jnp.float32),
            pltpu.VMEM((_CAND,), jnp.float32),
            pltpu.VMEM((_CAND,), jnp.int32),
            pltpu.VMEM((_RPW * 16,), jnp.float32),
            pltpu.VMEM((K,), jnp.int32),
            pltpu.VMEM((K,), jnp.int32),
            pltpu.VMEM((K, 128), jnp.float32),
            pltpu.VMEM((K, 128), jnp.float32),
            pltpu.VMEM((_RPW, 64), jnp.float32),
            pltpu.SemaphoreType.DMA,
            pltpu.SemaphoreType.DMA,
            pltpu.SemaphoreType.DMA,
        ],
        compiler_params=pltpu.CompilerParams(needs_layout_passes=False),
    )
    return run(ddf, tauf, h2f)


def kernel(xyz, points, W1, b1, W2, b2):
    x = xyz[:, :, 0]
    y = xyz[:, :, 1]
    z = xyz[:, :, 2]
    nx, ny, nz = _fps(x, y, z)
    new_xyz = jnp.stack([nx, ny, nz], axis=-1)

    xyz_t = jnp.transpose(xyz, (0, 2, 1))
    xyz8 = jnp.concatenate(
        [xyz_t, jnp.zeros((B, 5, N), jnp.float32)], axis=1)
    q8 = jnp.concatenate(
        [jnp.stack([nx, ny, nz], axis=-1),
         jnp.zeros((B, S, 5), jnp.float32)], axis=-1)
    w1a = jnp.concatenate([W1[:, :3], jnp.zeros((32, 5), jnp.float32)], axis=1)
    w1b = W1[:, 3:]
    dd, tau, h2 = _feat(xyz8, points, q8, w1a, w1b,
                        b1.reshape(32, 1), W2, b2.reshape(64, 1))

    ddf = dd.reshape(_ROWS, N)
    tauf = tau.reshape(_ROWS * 16)
    h2f = h2.reshape(B * N, 128)
    pooled = _sc_topk_gather_max(ddf, tauf, h2f)
    new_points = jnp.transpose(pooled.reshape(B, S, 64), (0, 2, 1))
    return new_xyz, new_points


# speedup vs baseline: 1.4497x; 1.4049x over previous
import jax
import jax.numpy as jnp
from jax import lax
from jax.experimental import pallas as pl
from jax.experimental.pallas import tpu as pltpu
from jax.experimental.pallas import tpu_sc as plsc

B = 8
N = 8192
S = 256
K = 32

_NC = 2
_NS = 16
_NW = _NC * _NS
_ROWS = B * S
_RPW = _ROWS // _NW
_CAND = 256
_VPR = N // 16


def _fps_body(x_ref, y_ref, z_ref, nx_ref, ny_ref, nz_ref):
    X = x_ref[...]
    Y = y_ref[...]
    Z = z_ref[...]
    col = lax.broadcasted_iota(jnp.int32, (B, N), 1)
    col_s = lax.broadcasted_iota(jnp.int32, (B, S), 1)

    def step(i, carry):
        dist, f, ax, ay, az = carry
        onehot = col == f
        cx = jnp.sum(jnp.where(onehot, X, 0.0), axis=1, keepdims=True)
        cy = jnp.sum(jnp.where(onehot, Y, 0.0), axis=1, keepdims=True)
        cz = jnp.sum(jnp.where(onehot, Z, 0.0), axis=1, keepdims=True)
        sel = col_s == i
        ax = jnp.where(sel, cx, ax)
        ay = jnp.where(sel, cy, ay)
        az = jnp.where(sel, cz, az)
        d = (X - cx) ** 2 + (Y - cy) ** 2 + (Z - cz) ** 2
        dist = jnp.minimum(dist, d)
        m = jnp.max(dist, axis=1, keepdims=True)
        f = jnp.min(jnp.where(dist == m, col, N), axis=1, keepdims=True)
        return dist, f, ax, ay, az

    zero_s = jnp.zeros((B, S), jnp.float32)
    init = (jnp.full((B, N), 1e10, jnp.float32),
            jnp.zeros((B, 1), jnp.int32), zero_s, zero_s, zero_s)
    _, _, ax, ay, az = lax.fori_loop(0, S, step, init)
    nx_ref[...] = ax
    ny_ref[...] = ay
    nz_ref[...] = az


def _fps(x, y, z):
    out = jax.ShapeDtypeStruct((B, S), jnp.float32)
    return pl.pallas_call(
        _fps_body,
        out_shape=[out, out, out],
    )(x, y, z)


def _feat_body(xyz8_ref, pts_ref, q8_ref, w1a_ref, w1b_ref, b1_ref,
               w2_ref, b2_ref, dd_ref, tau_ref, h2_ref):
    xyz8 = xyz8_ref[0]
    q8 = q8_ref[0]
    inner = jnp.dot(q8, xyz8, preferred_element_type=jnp.float32)
    x2 = jnp.sum(xyz8 * xyz8, axis=0, keepdims=True)
    q2 = jnp.sum(q8 * q8, axis=1, keepdims=True)
    dd = (q2 + x2) - 2.0 * inner
    dd_ref[0] = dd

    w = dd
    for width in (4096, 2048, 1024, 512, 256, 128):
        w = jnp.minimum(w[:, :width], w[:, width:2 * width])
    colg = lax.broadcasted_iota(jnp.int32, (S, 128), 1)

    def tsel(_, wc):
        m = jnp.min(wc, axis=1, keepdims=True)
        p = jnp.min(jnp.where(wc == m, colg, 128), axis=1, keepdims=True)
        return jnp.where(colg == p, jnp.inf, wc)

    w31 = lax.fori_loop(0, K - 1, tsel, w)
    tau = jnp.min(w31, axis=1, keepdims=True)
    tau_ref[0] = jnp.broadcast_to(tau, (S, 16))

    pts = pts_ref[0]
    h1 = (jnp.dot(w1a_ref[...], xyz8, preferred_element_type=jnp.float32)
          + jnp.dot(w1b_ref[...], pts, preferred_element_type=jnp.float32))
    h1 = jnp.maximum(h1 + b1_ref[...], 0.0)
    h2 = jnp.dot(w2_ref[...], h1, preferred_element_type=jnp.float32)
    h2 = jnp.maximum(h2 + b2_ref[...], 0.0)
    h2_ref[0, :, 0:64] = h2.T


def _feat(xyz8, pts, q8, w1a, w1b, b1c, w2, b2c):
    return pl.pallas_call(
        _feat_body,
        grid=(B,),
        in_specs=[
            pl.BlockSpec((1, 8, N), lambda b: (b, 0, 0)),
            pl.BlockSpec((1, 32, N), lambda b: (b, 0, 0)),
            pl.BlockSpec((1, S, 8), lambda b: (b, 0, 0)),
            pl.BlockSpec((32, 8), lambda b: (0, 0)),
            pl.BlockSpec((32, 32), lambda b: (0, 0)),
            pl.BlockSpec((32, 1), lambda b: (0, 0)),
            pl.BlockSpec((64, 32), lambda b: (0, 0)),
            pl.BlockSpec((64, 1), lambda b: (0, 0)),
        ],
        out_specs=[
            pl.BlockSpec((1, S, N), lambda b: (b, 0, 0)),
            pl.BlockSpec((1, S, 16), lambda b: (b, 0, 0)),
            pl.BlockSpec((1, N, 128), lambda b: (b, 0, 0)),
        ],
        out_shape=[
            jax.ShapeDtypeStruct((B, S, N), jnp.float32),
            jax.ShapeDtypeStruct((B, S, 16), jnp.float32),
            jax.ShapeDtypeStruct((B, N, 128), jnp.float32),
        ],
    )(xyz8, pts, q8, w1a, w1b, b1c, w2, b2c)


def _rev(x):
    return lax.rev(x, (0,))


def _minmax_kv(ak, av, bk, bv):
    m = ak <= bk
    return (jnp.where(m, ak, bk), jnp.where(m, av, bv),
            jnp.where(m, bk, ak), jnp.where(m, bv, av))


def _merge16(ak, av, bk, bv):
    brk, brv = _rev(bk), _rev(bv)
    lok, lov, hik, hiv = _minmax_kv(ak, av, brk, brv)
    lok, lov = plsc.sort_key_val(lok, lov)
    hik, hiv = plsc.sort_key_val(hik, hiv)
    return lok, lov, hik, hiv


def _merge32_keep32(x1k, x1v, x2k, x2v, y1k, y1v, y2k, y2v):
    ry1k, ry1v = _rev(y1k), _rev(y1v)
    ry2k, ry2v = _rev(y2k), _rev(y2v)
    lo1k, lo1v, _, _ = _minmax_kv(x1k, x1v, ry2k, ry2v)
    lo2k, lo2v, _, _ = _minmax_kv(x2k, x2v, ry1k, ry1v)
    m1k, m1v, m2k, m2v = _minmax_kv(lo1k, lo1v, lo2k, lo2v)
    m1k, m1v = plsc.sort_key_val(m1k, m1v)
    m2k, m2v = plsc.sort_key_val(m2k, m2v)
    return m1k, m1v, m2k, m2v


_RSEG = 8
_SEGW = N // _RSEG
_CAPR = _CAND // _RSEG


def _sc_topk_body(dd_hbm, tau_hbm, h2_hbm, out_hbm,
                  row_a, row_b, kb, vb, taub, idx_a, idx_b,
                  rows_a, rows_b, res_v, sem_a, sem_b, sem_g):
    cid = lax.axis_index("c")
    sid = lax.axis_index("s")
    wid = sid * _NC + cid
    base = wid * _RPW
    pltpu.sync_copy(tau_hbm.at[pl.ds(base * 16, _RPW * 16)], taub)
    pltpu.async_copy(dd_hbm.at[pl.ds(base, 1)], row_a, sem_a)
    pltpu.async_copy(dd_hbm.at[pl.ds(base + 1, 1)], row_b, sem_b)
    lane = lax.iota(jnp.int32, 16)
    inf16 = jnp.full((16,), jnp.inf, jnp.float32)

    def extract(rl, row_v):
        for i in range(_CAND // 16):
            kb[pl.ds(i * 16, 16)] = inf16
        tauv = taub[pl.ds(rl * 16, 16)]
        rb = ((base + rl) // S) * N

        def ext(j, carry):
            offs, ivecs = carry
            n_offs, n_ivecs = [], []
            for r in range(_RSEG):
                v = row_v[0, pl.ds(r * _SEGW + j * 16, 16)]
                msk = v <= tauv
                lim = (r + 1) * _CAPR - 16
                off_c = jnp.minimum(offs[r], lim)
                plsc.store_compressed(kb.at[pl.ds(off_c, 16)], v, mask=msk)
                plsc.store_compressed(vb.at[pl.ds(off_c, 16)], ivecs[r],
                                      mask=msk)
                cntv = plsc.all_reduce_population_count(msk)
                n_offs.append(offs[r] + cntv[0])
                n_ivecs.append(ivecs[r] + 16)
            return tuple(n_offs), tuple(n_ivecs)

        offs0 = tuple(jnp.int32(r * _CAPR) for r in range(_RSEG))
        ivecs0 = tuple(lane + rb + r * _SEGW for r in range(_RSEG))
        offs, _ = lax.fori_loop(0, _SEGW // 16, ext, (offs0, ivecs0))
        return offs

    def select(par_idxb, offs):
        def sortpair(s0):
            a_k, a_v = plsc.sort_key_val(kb[pl.ds(s0, 16)],
                                         vb[pl.ds(s0, 16)])
            b_k, b_v = plsc.sort_key_val(kb[pl.ds(s0 + 16, 16)],
                                         vb[pl.ds(s0 + 16, 16)])
            return _merge16(a_k, a_v, b_k, b_v)

        del offs
        pairs = [sortpair(r * _CAPR) for r in range(_RSEG)]
        while len(pairs) > 1:
            pairs = [_merge32_keep32(*pairs[i], *pairs[i + 1])
                     for i in range(0, len(pairs), 2)]
        bf = pairs[0]
        par_idxb[pl.ds(0, 16)] = bf[1]
        par_idxb[pl.ds(16, 16)] = bf[3]

    def reduce_into(r_prev, rows_v):
        for cb in range(4):
            sl = pl.ds(cb * 16, 16)
            acc = rows_v[0, sl]
            for r in range(1, K):
                acc = jnp.maximum(acc, rows_v[r, sl])
            res_v[r_prev, sl] = acc

    bufs = ((row_a, sem_a, idx_a, rows_a), (row_b, sem_b, idx_b, rows_b))

    def pair_loop(rp, _):
        for par in range(2):
            rl = rp * 2 + par
            row_v, sem, idxb, rows_v = bufs[par]
            o_idxb, o_rows = bufs[1 - par][2], bufs[1 - par][3]
            pltpu.make_async_copy(
                dd_hbm.at[pl.ds(base + rl, 1)], row_v, sem).wait()
            offs = extract(rl, row_v)

            @pl.when(rl + 2 < _RPW)
            def _prefetch():
                pltpu.async_copy(
                    dd_hbm.at[pl.ds(base + rl + 2, 1)], row_v, sem)
            if par == 1:
                pltpu.make_async_copy(
                    h2_hbm.at[o_idxb], o_rows, sem_g).wait()
                reduce_into(rl - 1, o_rows)
            else:
                @pl.when(rp > 0)
                def _drain():
                    pltpu.make_async_copy(
                        h2_hbm.at[o_idxb], o_rows, sem_g).wait()
                    reduce_into(rl - 1, o_rows)
            select(idxb, offs)
            pltpu.async_copy(h2_hbm.at[idxb], rows_v, sem_g)
        return 0

    lax.fori_loop(0, _RPW // 2, pair_loop, 0)
    pltpu.make_async_copy(h2_hbm.at[idx_b], rows_b, sem_g).wait()
    reduce_into(_RPW - 1, rows_b)
    pltpu.sync_copy(res_v, out_hbm.at[pl.ds(base, _RPW)])


def _sc_topk_gather_max(ddf, tauf, h2f):
    run = pl.kernel(
        _sc_topk_body,
        out_type=jax.ShapeDtypeStruct((_ROWS, 64), jnp.float32),
        mesh=plsc.VectorSubcoreMesh(
            core_axis_name="c", subcore_axis_name="s",
            num_cores=_NC, num_subcores=_NS),
        scratch_types=[
            pltpu.VMEM((1, N), jnp.float32),
            pltpu.VMEM((1, N), jnp.float32),
            pltpu.VMEM((_CAND,), jnp.float32),
            pltpu.VMEM((_CAND,), jnp.int32),
            pltpu.VMEM((_RPW * 16,), jnp.float32),
            pltpu.VMEM((K,), jnp.int32),
            pltpu.VMEM((K,), jnp.int32),
            pltpu.VMEM((K, 128), jnp.float32),
            pltpu.VMEM((K, 128), jnp.float32),
            pltpu.VMEM((_RPW, 64), jnp.float32),
            pltpu.SemaphoreType.DMA,
            pltpu.SemaphoreType.DMA,
            pltpu.SemaphoreType.DMA,
        ],
        compiler_params=pltpu.CompilerParams(needs_layout_passes=False),
    )
    return run(ddf, tauf, h2f)


def kernel(xyz, points, W1, b1, W2, b2):
    x = xyz[:, :, 0]
    y = xyz[:, :, 1]
    z = xyz[:, :, 2]
    nx, ny, nz = _fps(x, y, z)
    new_xyz = jnp.stack([nx, ny, nz], axis=-1)

    xyz_t = jnp.transpose(xyz, (0, 2, 1))
    xyz8 = jnp.concatenate(
        [xyz_t, jnp.zeros((B, 5, N), jnp.float32)], axis=1)
    q8 = jnp.concatenate(
        [jnp.stack([nx, ny, nz], axis=-1),
         jnp.zeros((B, S, 5), jnp.float32)], axis=-1)
    w1a = jnp.concatenate([W1[:, :3], jnp.zeros((32, 5), jnp.float32)], axis=1)
    w1b = W1[:, 3:]
    dd, tau, h2 = _feat(xyz8, points, q8, w1a, w1b,
                        b1.reshape(32, 1), W2, b2.reshape(64, 1))

    ddf = dd.reshape(_ROWS, N)
    tauf = tau.reshape(_ROWS * 16)
    h2f = h2.reshape(B * N, 128)
    pooled = _sc_topk_gather_max(ddf, tauf, h2f)
    new_points = jnp.transpose(pooled.reshape(B, S, 64), (0, 2, 1))
    return new_xyz, new_points


# speedup vs baseline: 1.4610x; 1.0077x over previous
import jax
import jax.numpy as jnp
from jax import lax
from jax.experimental import pallas as pl
from jax.experimental.pallas import tpu as pltpu
from jax.experimental.pallas import tpu_sc as plsc

B = 8
N = 8192
S = 256
K = 32

_NC = 2
_NS = 16
_NW = _NC * _NS
_ROWS = B * S
_RPW = _ROWS // _NW
_CAND = 256
_VPR = N // 16


def _fps_body(x_ref, y_ref, z_ref, nx_ref, ny_ref, nz_ref):
    X = x_ref[...]
    Y = y_ref[...]
    Z = z_ref[...]
    col = lax.broadcasted_iota(jnp.int32, (B, N), 1)
    col_s = lax.broadcasted_iota(jnp.int32, (B, S), 1)

    def step(i, carry):
        dist, f, ax, ay, az = carry
        onehot = col == f
        cx = jnp.sum(jnp.where(onehot, X, 0.0), axis=1, keepdims=True)
        cy = jnp.sum(jnp.where(onehot, Y, 0.0), axis=1, keepdims=True)
        cz = jnp.sum(jnp.where(onehot, Z, 0.0), axis=1, keepdims=True)
        sel = col_s == i
        ax = jnp.where(sel, cx, ax)
        ay = jnp.where(sel, cy, ay)
        az = jnp.where(sel, cz, az)
        d = (X - cx) ** 2 + (Y - cy) ** 2 + (Z - cz) ** 2
        dist = jnp.minimum(dist, d)
        m = jnp.max(dist, axis=1, keepdims=True)
        f = jnp.min(jnp.where(dist == m, col, N), axis=1, keepdims=True)
        return dist, f, ax, ay, az

    zero_s = jnp.zeros((B, S), jnp.float32)
    init = (jnp.full((B, N), 1e10, jnp.float32),
            jnp.zeros((B, 1), jnp.int32), zero_s, zero_s, zero_s)
    _, _, ax, ay, az = lax.fori_loop(0, S, step, init)
    nx_ref[...] = ax
    ny_ref[...] = ay
    nz_ref[...] = az


def _fps(x, y, z):
    out = jax.ShapeDtypeStruct((B, S), jnp.float32)
    return pl.pallas_call(
        _fps_body,
        out_shape=[out, out, out],
    )(x, y, z)


def _feat_body(xyz8_ref, pts_ref, q8_ref, w1a_ref, w1b_ref, b1_ref,
               w2_ref, b2_ref, dd_ref, tau_ref, h2_ref):
    xyz8 = xyz8_ref[0]
    q8 = q8_ref[0]
    inner = jnp.dot(q8, xyz8, preferred_element_type=jnp.float32)
    x2 = jnp.sum(xyz8 * xyz8, axis=0, keepdims=True)
    q2 = jnp.sum(q8 * q8, axis=1, keepdims=True)
    dd = (q2 + x2) - 2.0 * inner
    dd_ref[0] = dd

    w = dd
    for width in (4096, 2048, 1024, 512, 256, 128):
        w = jnp.minimum(w[:, :width], w[:, width:2 * width])
    colg = lax.broadcasted_iota(jnp.int32, (S, 128), 1)

    def tsel(_, wc):
        m = jnp.min(wc, axis=1, keepdims=True)
        p = jnp.min(jnp.where(wc == m, colg, 128), axis=1, keepdims=True)
        return jnp.where(colg == p, jnp.inf, wc)

    w31 = lax.fori_loop(0, K - 1, tsel, w)
    tau = jnp.min(w31, axis=1, keepdims=True)
    tau_ref[0] = jnp.broadcast_to(tau, (S, 16))

    pts = pts_ref[0]
    h1 = (jnp.dot(w1a_ref[...], xyz8, preferred_element_type=jnp.float32)
          + jnp.dot(w1b_ref[...], pts, preferred_element_type=jnp.float32))
    h1 = jnp.maximum(h1 + b1_ref[...], 0.0)
    h2 = jnp.dot(w2_ref[...], h1, preferred_element_type=jnp.float32)
    h2 = jnp.maximum(h2 + b2_ref[...], 0.0)
    h2_ref[0, :, 0:64] = h2.T


def _feat(xyz8, pts, q8, w1a, w1b, b1c, w2, b2c):
    return pl.pallas_call(
        _feat_body,
        grid=(B,),
        in_specs=[
            pl.BlockSpec((1, 8, N), lambda b: (b, 0, 0)),
            pl.BlockSpec((1, 32, N), lambda b: (b, 0, 0)),
            pl.BlockSpec((1, S, 8), lambda b: (b, 0, 0)),
            pl.BlockSpec((32, 8), lambda b: (0, 0)),
            pl.BlockSpec((32, 32), lambda b: (0, 0)),
            pl.BlockSpec((32, 1), lambda b: (0, 0)),
            pl.BlockSpec((64, 32), lambda b: (0, 0)),
            pl.BlockSpec((64, 1), lambda b: (0, 0)),
        ],
        out_specs=[
            pl.BlockSpec((1, S, N), lambda b: (b, 0, 0)),
            pl.BlockSpec((1, S, 16), lambda b: (b, 0, 0)),
            pl.BlockSpec((1, N, 128), lambda b: (b, 0, 0)),
        ],
        out_shape=[
            jax.ShapeDtypeStruct((B, S, N), jnp.float32),
            jax.ShapeDtypeStruct((B, S, 16), jnp.float32),
            jax.ShapeDtypeStruct((B, N, 128), jnp.float32),
        ],
    )(xyz8, pts, q8, w1a, w1b, b1c, w2, b2c)


def _rev(x):
    return lax.rev(x, (0,))


def _minmax_kv(ak, av, bk, bv):
    m = ak <= bk
    return (jnp.where(m, ak, bk), jnp.where(m, av, bv),
            jnp.where(m, bk, ak), jnp.where(m, bv, av))


def _merge16(ak, av, bk, bv):
    brk, brv = _rev(bk), _rev(bv)
    lok, lov, hik, hiv = _minmax_kv(ak, av, brk, brv)
    lok, lov = plsc.sort_key_val(lok, lov)
    hik, hiv = plsc.sort_key_val(hik, hiv)
    return lok, lov, hik, hiv


def _merge32_keep32(x1k, x1v, x2k, x2v, y1k, y1v, y2k, y2v):
    ry1k, ry1v = _rev(y1k), _rev(y1v)
    ry2k, ry2v = _rev(y2k), _rev(y2v)
    lo1k, lo1v, _, _ = _minmax_kv(x1k, x1v, ry2k, ry2v)
    lo2k, lo2v, _, _ = _minmax_kv(x2k, x2v, ry1k, ry1v)
    m1k, m1v, m2k, m2v = _minmax_kv(lo1k, lo1v, lo2k, lo2v)
    m1k, m1v = plsc.sort_key_val(m1k, m1v)
    m2k, m2v = plsc.sort_key_val(m2k, m2v)
    return m1k, m1v, m2k, m2v


_RSEG = 4
_UNROLL = 4
_SEGW = N // _RSEG
_CAPR = _CAND // _RSEG


def _sc_topk_body(dd_hbm, tau_hbm, h2_hbm, out_hbm,
                  row_a, row_b, kb, vb, taub, idx_a, idx_b,
                  rows_a, rows_b, res_v, sem_a, sem_b, sem_g):
    cid = lax.axis_index("c")
    sid = lax.axis_index("s")
    wid = sid * _NC + cid
    base = wid * _RPW
    pltpu.sync_copy(tau_hbm.at[pl.ds(base * 16, _RPW * 16)], taub)
    pltpu.async_copy(dd_hbm.at[pl.ds(base, 1)], row_a, sem_a)
    pltpu.async_copy(dd_hbm.at[pl.ds(base + 1, 1)], row_b, sem_b)
    lane = lax.iota(jnp.int32, 16)
    inf16 = jnp.full((16,), jnp.inf, jnp.float32)

    def extract(rl, row_v):
        for i in range(_CAND // 16):
            kb[pl.ds(i * 16, 16)] = inf16
        tauv = taub[pl.ds(rl * 16, 16)]
        rb = ((base + rl) // S) * N

        def ext(j, carry):
            offs, ivecs = carry
            offs = list(offs)
            ivecs = list(ivecs)
            for u in range(_UNROLL):
                for r in range(_RSEG):
                    v = row_v[0, pl.ds(r * _SEGW + j * 16 * _UNROLL
                                       + u * 16, 16)]
                    msk = v <= tauv
                    lim = (r + 1) * _CAPR - 16
                    off_c = jnp.minimum(offs[r], lim)
                    plsc.store_compressed(kb.at[pl.ds(off_c, 16)], v,
                                          mask=msk)
                    plsc.store_compressed(vb.at[pl.ds(off_c, 16)], ivecs[r],
                                          mask=msk)
                    cntv = plsc.all_reduce_population_count(msk)
                    offs[r] = offs[r] + cntv[0]
                    ivecs[r] = ivecs[r] + 16
            return tuple(offs), tuple(ivecs)

        offs0 = tuple(jnp.int32(r * _CAPR) for r in range(_RSEG))
        ivecs0 = tuple(lane + rb + r * _SEGW for r in range(_RSEG))
        offs, _ = lax.fori_loop(0, _SEGW // (16 * _UNROLL), ext,
                                (offs0, ivecs0))
        return offs

    def select(par_idxb, offs):
        def sortpair(s0):
            a_k, a_v = plsc.sort_key_val(kb[pl.ds(s0, 16)],
                                         vb[pl.ds(s0, 16)])
            b_k, b_v = plsc.sort_key_val(kb[pl.ds(s0 + 16, 16)],
                                         vb[pl.ds(s0 + 16, 16)])
            return _merge16(a_k, a_v, b_k, b_v)

        bf = sortpair(0)
        for r in range(1, _RSEG):
            bf = _merge32_keep32(*bf, *sortpair(r * _CAPR))
        for r in range(_RSEG):
            cnt_r = offs[r] - r * _CAPR
            bf = lax.cond(
                cnt_r > 32,
                lambda bf=bf, r=r: _merge32_keep32(
                    *bf, *sortpair(r * _CAPR + 32)),
                lambda bf=bf: bf)
        par_idxb[pl.ds(0, 16)] = bf[1]
        par_idxb[pl.ds(16, 16)] = bf[3]

    def reduce_into(r_prev, rows_v):
        for cb in range(4):
            sl = pl.ds(cb * 16, 16)
            acc = rows_v[0, sl]
            for r in range(1, K):
                acc = jnp.maximum(acc, rows_v[r, sl])
            res_v[r_prev, sl] = acc

    bufs = ((row_a, sem_a, idx_a, rows_a), (row_b, sem_b, idx_b, rows_b))

    def pair_loop(rp, _):
        for par in range(2):
            rl = rp * 2 + par
            row_v, sem, idxb, rows_v = bufs[par]
            o_idxb, o_rows = bufs[1 - par][2], bufs[1 - par][3]
            pltpu.make_async_copy(
                dd_hbm.at[pl.ds(base + rl, 1)], row_v, sem).wait()
            offs = extract(rl, row_v)

            @pl.when(rl + 2 < _RPW)
            def _prefetch():
                pltpu.async_copy(
                    dd_hbm.at[pl.ds(base + rl + 2, 1)], row_v, sem)
            if par == 1:
                pltpu.make_async_copy(
                    h2_hbm.at[o_idxb], o_rows, sem_g).wait()
                reduce_into(rl - 1, o_rows)
            else:
                @pl.when(rp > 0)
                def _drain():
                    pltpu.make_async_copy(
                        h2_hbm.at[o_idxb], o_rows, sem_g).wait()
                    reduce_into(rl - 1, o_rows)
            select(idxb, offs)
            pltpu.async_copy(h2_hbm.at[idxb], rows_v, sem_g)
        return 0

    lax.fori_loop(0, _RPW // 2, pair_loop, 0)
    pltpu.make_async_copy(h2_hbm.at[idx_b], rows_b, sem_g).wait()
    reduce_into(_RPW - 1, rows_b)
    pltpu.sync_copy(res_v, out_hbm.at[pl.ds(base, _RPW)])


def _sc_topk_gather_max(ddf, tauf, h2f):
    run = pl.kernel(
        _sc_topk_body,
        out_type=jax.ShapeDtypeStruct((_ROWS, 64), jnp.float32),
        mesh=plsc.VectorSubcoreMesh(
            core_axis_name="c", subcore_axis_name="s",
            num_cores=_NC, num_subcores=_NS),
        scratch_types=[
            pltpu.VMEM((1, N), jnp.float32),
            pltpu.VMEM((1, N), jnp.float32),
            pltpu.VMEM((_CAND,), jnp.float32),
            pltpu.VMEM((_CAND,), jnp.int32),
            pltpu.VMEM((_RPW * 16,), jnp.float32),
            pltpu.VMEM((K,), jnp.int32),
            pltpu.VMEM((K,), jnp.int32),
            pltpu.VMEM((K, 128), jnp.float32),
            pltpu.VMEM((K, 128), jnp.float32),
            pltpu.VMEM((_RPW, 64), jnp.float32),
            pltpu.SemaphoreType.DMA,
            pltpu.SemaphoreType.DMA,
            pltpu.SemaphoreType.DMA,
        ],
        compiler_params=pltpu.CompilerParams(needs_layout_passes=False),
    )
    return run(ddf, tauf, h2f)


def kernel(xyz, points, W1, b1, W2, b2):
    x = xyz[:, :, 0]
    y = xyz[:, :, 1]
    z = xyz[:, :, 2]
    nx, ny, nz = _fps(x, y, z)
    new_xyz = jnp.stack([nx, ny, nz], axis=-1)

    xyz_t = jnp.transpose(xyz, (0, 2, 1))
    xyz8 = jnp.concatenate(
        [xyz_t, jnp.zeros((B, 5, N), jnp.float32)], axis=1)
    q8 = jnp.concatenate(
        [jnp.stack([nx, ny, nz], axis=-1),
         jnp.zeros((B, S, 5), jnp.float32)], axis=-1)
    w1a = jnp.concatenate([W1[:, :3], jnp.zeros((32, 5), jnp.float32)], axis=1)
    w1b = W1[:, 3:]
    dd, tau, h2 = _feat(xyz8, points, q8, w1a, w1b,
                        b1.reshape(32, 1), W2, b2.reshape(64, 1))

    ddf = dd.reshape(_ROWS, N)
    tauf = tau.reshape(_ROWS * 16)
    h2f = h2.reshape(B * N, 128)
    pooled = _sc_topk_gather_max(ddf, tauf, h2f)
    new_points = jnp.transpose(pooled.reshape(B, S, 64), (0, 2, 1))
    return new_xyz, new_points
